# baseline (speedup 1.0000x reference)
import functools

import jax
import jax.numpy as jnp
import numpy as np
from jax import lax
from jax.experimental import pallas as pl
from jax.experimental.pallas import tpu as pltpu
from jax.experimental.pallas import tpu_sc as plsc

DEPTH = 300
DEPTH_PAD = 304
MAX_LENGTH = 200
SCALE = float(np.sqrt(float(DEPTH)))

NUM_CORES = 2
NUM_SUBCORES = 16
NUM_WORKERS = NUM_CORES * NUM_SUBCORES
CHUNK = 80
N_FULL = DEPTH // 16
TAIL = DEPTH - 16


def _positional_encoding_np(length, depth):
    half = depth / 2
    positions = np.arange(length)[:, np.newaxis]
    depths = np.arange(half)[np.newaxis, :] / half
    angle_rates = 1 / 10000 ** depths
    angle_rads = positions * angle_rates
    return np.concatenate(
        [np.sin(angle_rads), np.cos(angle_rads)], axis=-1
    ).astype(np.float32)


_POS_NP = np.zeros((MAX_LENGTH, DEPTH_PAD), np.float32)
_POS_NP[:, :DEPTH] = _positional_encoding_np(MAX_LENGTH, DEPTH)


@functools.partial(jax.jit, static_argnames=("n_rows", "length"))
def _seq_embedding(seq_flat, pos, table, n_rows, length):
    table_pad = jnp.pad(table, ((0, 0), (0, DEPTH_PAD - table.shape[1])))
    rows_per_w = n_rows // NUM_WORKERS
    n_chunks = rows_per_w // CHUNK
    mesh = plsc.VectorSubcoreMesh(core_axis_name="c", subcore_axis_name="s")

    @functools.partial(
        pl.kernel,
        mesh=mesh,
        compiler_params=pltpu.CompilerParams(use_tc_tiling_on_sc=False),
        out_type=jax.ShapeDtypeStruct((n_rows, DEPTH), jnp.float32),
        scratch_types=[
            pltpu.VMEM((CHUNK,), jnp.int32),
            pltpu.VMEM((CHUNK, DEPTH_PAD), jnp.float32),
            pltpu.VMEM((CHUNK, DEPTH), jnp.float32),
            pltpu.VMEM((length, DEPTH_PAD), jnp.float32),
            pltpu.SemaphoreType.DMA,
        ],
    )
    def body(seq_hbm, pos_hbm, table_hbm, out_hbm, idx_v, rows_v, out_v,
             pos_v, sem):
        wid = lax.axis_index("s") * NUM_CORES + lax.axis_index("c")
        pltpu.sync_copy(pos_hbm, pos_v)
        base_w = wid * rows_per_w

        def chunk_body(t, carry):
            base = base_w + t * CHUNK
            pltpu.sync_copy(seq_hbm.at[pl.ds(base, CHUNK)], idx_v)
            pltpu.async_copy(table_hbm.at[idx_v], rows_v, sem).wait()

            def row_body(r, carry2):
                pr = lax.rem(base + r, length)
                for i in range(N_FULL):
                    sl = pl.ds(i * 16, 16)
                    out_v[r, sl] = rows_v[r, sl] * SCALE + pos_v[pr, sl]
                sl = pl.ds(TAIL, 16)
                out_v[r, sl] = rows_v[r, sl] * SCALE + pos_v[pr, sl]
                return carry2

            lax.fori_loop(0, CHUNK, row_body, 0)
            pltpu.sync_copy(out_v, out_hbm.at[pl.ds(base, CHUNK)])
            return carry

        lax.fori_loop(0, n_chunks, chunk_body, 0)

    return body(seq_flat, pos, table_pad)


def kernel(seq, table):
    batch, length = seq.shape
    n_rows = batch * length
    seq_flat = seq.reshape(n_rows).astype(jnp.int32)
    pos = jnp.asarray(_POS_NP[:length])
    out = _seq_embedding(seq_flat, pos, table, n_rows, length)
    return out.reshape(batch, length, DEPTH)


# speedup vs baseline: 1.4515x; 1.4515x over previous
import functools

import jax
import jax.numpy as jnp
import numpy as np
from jax import lax
from jax.experimental import pallas as pl
from jax.experimental.pallas import tpu as pltpu
from jax.experimental.pallas import tpu_sc as plsc

DEPTH = 300
MAX_LENGTH = 200
SCALE = float(np.sqrt(float(DEPTH)))

NUM_CORES = 2
NUM_SUBCORES = 16
NUM_WORKERS = NUM_CORES * NUM_SUBCORES
D_LO = DEPTH // NUM_WORKERS
D_EXTRA = DEPTH - D_LO * NUM_WORKERS
D_HI = D_LO + 1
L_BLK = 8


def _positional_encoding_np(length, depth):
    half = depth / 2
    positions = np.arange(length)[:, np.newaxis]
    depths = np.arange(half)[np.newaxis, :] / half
    angle_rates = 1 / 10000 ** depths
    angle_rads = positions * angle_rates
    return np.concatenate(
        [np.sin(angle_rads), np.cos(angle_rads)], axis=-1
    ).astype(np.float32)


_POS_REP = np.ascontiguousarray(
    np.broadcast_to(
        _positional_encoding_np(MAX_LENGTH, DEPTH).T[:, :, None],
        (DEPTH, MAX_LENGTH, 16),
    )
).reshape(-1)


@functools.partial(jax.jit, static_argnames=("n_vocab", "length", "batch"))
def _seq_embedding(seq_t, pos_rep, table_t, n_vocab, length, batch):
    n_blk = length // L_BLK
    n_slices = batch // 16
    mesh = plsc.VectorSubcoreMesh(core_axis_name="c", subcore_axis_name="s")

    @functools.partial(
        pl.kernel,
        mesh=mesh,
        compiler_params=pltpu.CompilerParams(
            use_tc_tiling_on_sc=False, needs_layout_passes=False),
        out_type=jax.ShapeDtypeStruct((DEPTH * length * batch,), jnp.float32),
        scratch_types=[
            pltpu.VMEM((n_vocab,), jnp.float32),
            pltpu.VMEM((L_BLK * batch,), jnp.int32),
            pltpu.VMEM((batch,), jnp.float32),
            pltpu.VMEM((batch,), jnp.float32),
            pltpu.VMEM((length * 16,), jnp.float32),
            pltpu.VMEM_SHARED((length * batch,), jnp.int32),
            pltpu.SemaphoreType.DMA,
            pltpu.SemaphoreType.DMA,
        ],
    )
    def body(seq_hbm, pos_hbm, table_hbm, out_hbm, row_v, seq_v, out_v0,
             out_v1, pos_v, seq_sh, sem0, sem1):
        cid = lax.axis_index("c")
        sid = lax.axis_index("s")
        wid = sid * NUM_CORES + cid

        @pl.when(sid == 0)
        def _stage():
            pltpu.sync_copy(seq_hbm, seq_sh)

        plsc.subcore_barrier()

        n_d = jnp.where(wid < D_EXTRA, D_HI, D_LO)
        d0 = jnp.where(
            wid < D_EXTRA,
            wid * D_HI,
            D_EXTRA * D_HI + (wid - D_EXTRA) * D_LO,
        )
        out_bufs = (out_v0, out_v1)
        sems = (sem0, sem1)

        def d_body(j, carry):
            d = d0 + j

            @pl.when(j < n_d)
            def _work():
                pltpu.sync_copy(table_hbm.at[pl.ds(d * n_vocab, n_vocab)],
                                row_v)
                pltpu.sync_copy(pos_hbm.at[pl.ds(d * (length * 16),
                                                 length * 16)], pos_v)
                out_base = d * (length * batch)

                def blk_body(bk, carry2):
                    l0 = bk * L_BLK
                    pltpu.sync_copy(seq_sh.at[pl.ds(l0 * batch,
                                                    L_BLK * batch)], seq_v)
                    for li in range(L_BLK):
                        l = l0 + li
                        p = li % 2
                        buf = out_bufs[p]
                        dst = out_hbm.at[pl.ds(out_base + l * batch, batch)]
                        if li >= 2:
                            pltpu.make_async_copy(buf, dst, sems[p]).wait()
                        else:
                            @pl.when(bk > 0)
                            def _drain():
                                pltpu.make_async_copy(buf, dst,
                                                      sems[p]).wait()
                        pos_vec = pos_v[pl.ds(l * 16, 16)]

                        def sl_body(s, carry3):
                            idx16 = seq_v[pl.ds(li * batch + s * 16, 16)]
                            g = plsc.load_gather(row_v, [idx16])
                            buf[pl.ds(s * 16, 16)] = g * SCALE + pos_vec
                            return carry3

                        lax.fori_loop(0, n_slices, sl_body, 0)
                        pltpu.async_copy(buf, dst, sems[p])
                    return carry2

                lax.fori_loop(0, n_blk, blk_body, 0)
                for p in range(2):
                    l_last = length - 2 + p
                    dst = out_hbm.at[pl.ds(out_base + l_last * batch, batch)]
                    pltpu.make_async_copy(out_bufs[p ^ (length % 2)], dst,
                                          sems[p ^ (length % 2)]).wait()

            return carry

        lax.fori_loop(0, D_HI, d_body, 0)

    return body(seq_t, pos_rep, table_t)


def kernel(seq, table):
    batch, length = seq.shape
    n_vocab, depth = table.shape
    seq_t = seq.T.reshape(-1).astype(jnp.int32)
    table_t = table.T.reshape(-1)
    pos_rep = jnp.asarray(
        _POS_REP.reshape(DEPTH, MAX_LENGTH, 16)[:, :length].reshape(-1))
    out_flat = _seq_embedding(seq_t, pos_rep, table_t, n_vocab, length, batch)
    return out_flat.reshape(depth, length, batch).transpose(2, 1, 0)


# speedup vs baseline: 2.8669x; 1.9751x over previous
import functools

import jax
import jax.numpy as jnp
import numpy as np
from jax import lax
from jax.experimental import pallas as pl
from jax.experimental.pallas import tpu as pltpu
from jax.experimental.pallas import tpu_sc as plsc

DEPTH = 300
MAX_LENGTH = 200
SCALE = float(np.sqrt(float(DEPTH)))

NUM_CORES = 2
NUM_SUBCORES = 16
NUM_WORKERS = NUM_CORES * NUM_SUBCORES
D_LO = DEPTH // NUM_WORKERS
D_EXTRA = DEPTH - D_LO * NUM_WORKERS
D_HI = D_LO + 1
L_BLK = 8


def _positional_encoding_np(length, depth):
    half = depth / 2
    positions = np.arange(length)[:, np.newaxis]
    depths = np.arange(half)[np.newaxis, :] / half
    angle_rates = 1 / 10000 ** depths
    angle_rads = positions * angle_rates
    return np.concatenate(
        [np.sin(angle_rads), np.cos(angle_rads)], axis=-1
    ).astype(np.float32)


_POS_REP = np.ascontiguousarray(
    np.broadcast_to(
        _positional_encoding_np(MAX_LENGTH, DEPTH).T[:, :, None],
        (DEPTH, MAX_LENGTH, 16),
    )
)


@functools.partial(jax.jit, static_argnames=("n_vocab", "length", "batch"))
def _seq_embedding(seq_t, pos_rep, table_t, n_vocab, length, batch):
    n_blk = length // L_BLK
    n_slices = batch // 16
    blk_elems = L_BLK * batch
    mesh = plsc.VectorSubcoreMesh(core_axis_name="c", subcore_axis_name="s")

    @functools.partial(
        pl.kernel,
        mesh=mesh,
        compiler_params=pltpu.CompilerParams(
            use_tc_tiling_on_sc=False, needs_layout_passes=False),
        out_type=jax.ShapeDtypeStruct((DEPTH * length * batch,), jnp.float32),
        scratch_types=[
            pltpu.VMEM((n_vocab,), jnp.float32),
            pltpu.VMEM((L_BLK * batch,), jnp.int32),
            pltpu.VMEM((blk_elems,), jnp.float32),
            pltpu.VMEM((blk_elems,), jnp.float32),
            pltpu.VMEM((length * 16,), jnp.float32),
            pltpu.SemaphoreType.DMA,
            pltpu.SemaphoreType.DMA,
        ],
    )
    def body(seq_hbm, pos_hbm, table_hbm, out_hbm, row_v, seq_v, out_v0,
             out_v1, pos_v, sem0, sem1):
        cid = lax.axis_index("c")
        sid = lax.axis_index("s")
        wid = sid * NUM_CORES + cid

        n_d = jnp.where(wid < D_EXTRA, D_HI, D_LO)
        d0 = jnp.where(
            wid < D_EXTRA,
            wid * D_HI,
            D_EXTRA * D_HI + (wid - D_EXTRA) * D_LO,
        )
        out_bufs = (out_v0, out_v1)
        sems = (sem0, sem1)

        def run_block(bk, d, buf, sem, first_use):
            out_base = d * (length * batch)
            dst = out_hbm.at[pl.ds(out_base + bk * blk_elems, blk_elems)]
            if first_use is None:
                pltpu.make_async_copy(buf, dst, sem).wait()
            else:
                @pl.when(jnp.logical_not(first_use))
                def _drain():
                    pltpu.make_async_copy(buf, dst, sem).wait()
            pltpu.sync_copy(
                seq_hbm.at[pl.ds(bk * (L_BLK * batch), L_BLK * batch)], seq_v)
            for li in range(L_BLK):
                l = bk * L_BLK + li
                pos_vec = pos_v[pl.ds(l * 16, 16)]

                @plsc.parallel_loop(0, n_slices, unroll=4)
                def _slices(s):
                    idx16 = seq_v[pl.ds(li * batch + s * 16, 16)]
                    g = plsc.load_gather(row_v, [idx16])
                    off = ((s >> 3) * (L_BLK * 128) + li * 128
                           + (s & 7) * 16)
                    buf[pl.ds(off, 16)] = g * SCALE + pos_vec

            pltpu.async_copy(buf, dst, sem)

        def d_body(j, carry):
            d = d0 + j

            @pl.when(j < n_d)
            def _work():
                pltpu.sync_copy(table_hbm.at[pl.ds(d * n_vocab, n_vocab)],
                                row_v)
                pltpu.sync_copy(pos_hbm.at[pl.ds(d * (length * 16),
                                                 length * 16)], pos_v)

                def pair_body(t, carry2):
                    for half in range(2):
                        bk = t * 2 + half
                        run_block(bk, d, out_bufs[half], sems[half],
                                  first_use=jnp.logical_and(j == 0, t == 0))
                    return carry2

                lax.fori_loop(0, n_blk // 2, pair_body, 0)
                if n_blk % 2:
                    run_block(n_blk - 1, d, out_bufs[0], sems[0],
                              first_use=None)

            return carry

        lax.fori_loop(0, D_HI, d_body, 0)
        d_last = d0 + n_d - 1
        for p in range(2):
            bk_last = n_blk - 2 + p if n_blk % 2 == 0 else (
                n_blk - 1 - p)
            dst = out_hbm.at[pl.ds(d_last * (length * batch)
                                   + bk_last * blk_elems, blk_elems)]
            pltpu.make_async_copy(out_bufs[p if n_blk % 2 == 0 else p],
                                  dst, sems[p]).wait()

    return body(seq_t, pos_rep, table_t)


def kernel(seq, table):
    batch, length = seq.shape
    n_vocab, depth = table.shape
    seq_t = seq.T.reshape(-1).astype(jnp.int32)
    table_t = table.T.reshape(-1)
    pos_rep = jnp.asarray(_POS_REP[:, :length].reshape(-1))
    out_flat = _seq_embedding(seq_t, pos_rep, table_t, n_vocab, length, batch)
    out5 = out_flat.reshape(depth, length // L_BLK, batch // 128, L_BLK, 128)
    return out5.transpose(2, 4, 1, 3, 0).reshape(batch, length, depth)
